# baseline (speedup 1.0000x reference)
import functools

import jax
import jax.numpy as jnp
from jax import lax
from jax.experimental import pallas as pl
from jax.experimental.pallas import tpu as pltpu
from jax.experimental.pallas import tpu_sc as plsc

TPB = 17
B, S, V, D = 1024, 340, 100000, 64
NC, NS = 2, 16
NW = NC * NS
N = B * S
RPW = N // NW
J = RPW // 128
APW = RPW // TPB
AJ = APW // 128


NBUF = 4
SPAD = 344
OUTR = B * SPAD * 2


def _sc_body(tok_hbm, tab_hbm, out_hbm,
             tok_v, act_tok, act_out, phys_idx, rows_v, act_rows,
             gsem0, gsem1, gsem2, gsem3, osem0, osem1, osem2, osem3, asem):
    gsem = (gsem0, gsem1, gsem2, gsem3)
    osem = (osem0, osem1, osem2, osem3)
    wid = lax.axis_index("s") * NC + lax.axis_index("c")
    base = wid * RPW
    lane = lax.iota(jnp.int32, 16)

    pltpu.sync_copy(tok_hbm.at[pl.ds(base, RPW)], tok_v)

    def ibody(j, carry):
        b0, s0 = carry
        pbase = 688 * b0 + 2 * s0
        for jj in range(8):
            i16 = jj * 16 + lane
            s_v = s0 + i16
            phys = pbase + 2 * i16 + lax.select(
                s_v >= S, jnp.full((16,), 8, jnp.int32),
                jnp.zeros((16,), jnp.int32))
            phys_idx[j, pl.ds(jj * 16, 16)] = phys
        s0n = s0 + 128
        wrap = (s0n >= S).astype(jnp.int32)
        return (b0 + wrap, s0n - S * wrap)

    lax.fori_loop(0, J, ibody, (32 * wid, jnp.int32(0)))

    def start_gather(j, b):
        idx = tok_v.at[pl.ds(j * 128, 128)]
        pltpu.async_copy(tab_hbm.at[idx], rows_v.at[b], gsem[b])

    def start_outcopy(j, b):
        pltpu.async_copy(rows_v.at[b], out_hbm.at[phys_idx.at[j]], osem[b])

    def wait_gather(b):
        pltpu.make_async_copy(out_hbm.at[pl.ds(0, 128)], rows_v.at[b],
                              gsem[b]).wait()

    def wait_outcopy(b):
        pltpu.make_async_copy(rows_v.at[b], out_hbm.at[pl.ds(0, 128)],
                              osem[b]).wait()

    for b in range(NBUF):
        start_gather(b, b)

    def grp(g, carry):
        for b in range(NBUF):
            wait_gather(b)
            start_outcopy(g * NBUF + b, b)
        for b in range(NBUF):
            wait_outcopy(b)
            start_gather((g + 1) * NBUF + b, b)
        return carry

    lax.fori_loop(0, J // NBUF - 1, grp, 0)
    for b in range(NBUF):
        wait_gather(b)
        start_outcopy((J // NBUF - 1) * NBUF + b, b)
    wait_outcopy(0)
    start_gather(J - 1, 0)
    wait_gather(0)
    start_outcopy(J - 1, 0)
    for b in range(1, NBUF):
        wait_outcopy(b)
    wait_outcopy(0)

    def xbody(j, carry):
        pb = j * (16 * TPB)
        acc = tok_v[pl.ds(pb + 16, 16)]
        for i in range(1, 16):
            v = tok_v[pl.ds(pb + 16 * (i + 1), 16)]
            acc = lax.select(lane == i, v, acc)
        act_tok[pl.ds(j * 16, 16)] = acc + V
        return carry

    lax.fori_loop(0, APW // 16, xbody, 0)

    for jj in range(APW // 16):
        p = jj * (16 * TPB) + TPB * lane + (TPB - 1)
        t0 = (jj * (16 * TPB) + (TPB - 1)) // S
        vals = 22016 * wid + 8 * t0 + 2 * p + lax.select(
            p >= S * (t0 + 1), jnp.full((16,), 8, jnp.int32),
            jnp.zeros((16,), jnp.int32))
        act_out[jj // 8, pl.ds((jj % 8) * 16, 16)] = vals

    for a in range(AJ):
        idx = act_tok.at[pl.ds(a * 128, 128)]
        pltpu.async_copy(tab_hbm.at[idx], act_rows.at[a], asem).wait()
    for a in range(AJ):
        pltpu.async_copy(act_rows.at[a], out_hbm.at[act_out.at[a]], asem).wait()


@jax.jit
def kernel(tokens, obs_table, act_table):
    tok1d = tokens.reshape(N)
    table = jnp.concatenate([obs_table, act_table], axis=0)
    call = pl.kernel(
        _sc_body,
        out_type=jax.ShapeDtypeStruct((OUTR, D), jnp.float32),
        mesh=plsc.VectorSubcoreMesh(core_axis_name="c", subcore_axis_name="s"),
        compiler_params=pltpu.CompilerParams(use_tc_tiling_on_sc=False),
        scratch_types=[
            pltpu.VMEM((RPW,), jnp.int32),
            pltpu.VMEM((APW,), jnp.int32),
            pltpu.VMEM((AJ, 128), jnp.int32),
            pltpu.VMEM((J, 128), jnp.int32),
            pltpu.VMEM((NBUF, 128, D), jnp.float32),
            pltpu.VMEM((AJ, 128, D), jnp.float32),
        ] + [pltpu.SemaphoreType.DMA] * (2 * NBUF + 1),
    )
    out = call(tok1d, table)
    return out.reshape(B, SPAD, 2 * D)[:, :S, :D]


# speedup vs baseline: 1.2822x; 1.2822x over previous
import functools

import jax
import jax.numpy as jnp
from jax import lax
from jax.experimental import pallas as pl
from jax.experimental.pallas import tpu as pltpu
from jax.experimental.pallas import tpu_sc as plsc

TPB = 17
B, S, V, D = 1024, 340, 100000, 64
NC, NS = 2, 16
NW = NC * NS
N = B * S
RPW = N // NW
J = RPW // 128
APW = RPW // TPB
AJ = APW // 128


NBUF = 6
SPAD = 344
OUTR = B * SPAD * 2


def _sc_body(tok_hbm, obs_hbm, act_hbm, out_hbm,
             tok_v, act_tok, act_out, phys_idx, rows_v, act_rows,
             gsem0, gsem1, gsem2, gsem3, gsem4, gsem5,
             osem0, osem1, osem2, osem3, osem4, osem5, asem):
    gsem = (gsem0, gsem1, gsem2, gsem3, gsem4, gsem5)
    osem = (osem0, osem1, osem2, osem3, osem4, osem5)
    wid = lax.axis_index("s") * NC + lax.axis_index("c")
    base = wid * RPW
    lane = lax.iota(jnp.int32, 16)

    pltpu.sync_copy(tok_hbm.at[pl.ds(base, RPW)], tok_v)

    def start_gather(j, b):
        idx = tok_v.at[pl.ds(j * 128, 128)]
        pltpu.async_copy(obs_hbm.at[idx], rows_v.at[b], gsem[b])

    def start_outcopy(j, b):
        pltpu.async_copy(rows_v.at[b], out_hbm.at[phys_idx.at[j]], osem[b])

    def wait_gather(b):
        pltpu.make_async_copy(out_hbm.at[pl.ds(0, 128)], rows_v.at[b],
                              gsem[b]).wait()

    def wait_outcopy(b):
        pltpu.make_async_copy(rows_v.at[b], out_hbm.at[pl.ds(0, 128)],
                              osem[b]).wait()

    for b in range(NBUF):
        start_gather(b, b)

    def ibody(j, carry):
        b0, s0 = carry
        pbase = 688 * b0 + 2 * s0
        for jj in range(8):
            i16 = jj * 16 + lane
            s_v = s0 + i16
            phys = pbase + 2 * i16 + lax.select(
                s_v >= S, jnp.full((16,), 8, jnp.int32),
                jnp.zeros((16,), jnp.int32))
            phys_idx[j, pl.ds(jj * 16, 16)] = phys
        s0n = s0 + 128
        wrap = (s0n >= S).astype(jnp.int32)
        return (b0 + wrap, s0n - S * wrap)

    lax.fori_loop(0, J, ibody, (32 * wid, jnp.int32(0)))

    def xbody(j, carry):
        pb = j * (16 * TPB)
        acc = tok_v[pl.ds(pb + 16, 16)]
        for i in range(1, 16):
            v = tok_v[pl.ds(pb + 16 * (i + 1), 16)]
            acc = lax.select(lane == i, v, acc)
        act_tok[pl.ds(j * 16, 16)] = acc
        return carry

    lax.fori_loop(0, APW // 16, xbody, 0)

    for jj in range(APW // 16):
        p = jj * (16 * TPB) + TPB * lane + (TPB - 1)
        t0 = (jj * (16 * TPB) + (TPB - 1)) // S
        vals = 22016 * wid + 8 * t0 + 2 * p + lax.select(
            p >= S * (t0 + 1), jnp.full((16,), 8, jnp.int32),
            jnp.zeros((16,), jnp.int32))
        act_out[jj // 8, pl.ds((jj % 8) * 16, 16)] = vals

    def grp(g, carry):
        for b in range(NBUF):
            wait_gather(b)
            start_outcopy(g * NBUF + b, b)
        for b in range(NBUF):
            wait_outcopy(b)
            start_gather((g + 1) * NBUF + b, b)
        return carry

    lax.fori_loop(0, J // NBUF - 1, grp, 0)
    for b in range(NBUF):
        wait_gather(b)
        start_outcopy((J // NBUF - 1) * NBUF + b, b)
    wait_outcopy(0)
    start_gather(J - 1, 0)
    wait_gather(0)
    start_outcopy(J - 1, 0)
    for b in range(1, NBUF):
        wait_outcopy(b)
    wait_outcopy(0)

    for a in range(AJ):
        idx = act_tok.at[pl.ds(a * 128, 128)]
        pltpu.async_copy(act_hbm.at[idx], act_rows.at[a], asem)
    for a in range(AJ):
        pltpu.make_async_copy(out_hbm.at[pl.ds(0, 128)], act_rows.at[a],
                              asem).wait()
    for a in range(AJ):
        pltpu.async_copy(act_rows.at[a], out_hbm.at[act_out.at[a]], asem)
    for a in range(AJ):
        pltpu.make_async_copy(act_rows.at[a], out_hbm.at[pl.ds(0, 128)],
                              asem).wait()


@jax.jit
def kernel(tokens, obs_table, act_table):
    tok1d = tokens.reshape(N)
    call = pl.kernel(
        _sc_body,
        out_type=jax.ShapeDtypeStruct((OUTR, D), jnp.float32),
        mesh=plsc.VectorSubcoreMesh(core_axis_name="c", subcore_axis_name="s"),
        compiler_params=pltpu.CompilerParams(use_tc_tiling_on_sc=False),
        scratch_types=[
            pltpu.VMEM((RPW,), jnp.int32),
            pltpu.VMEM((APW,), jnp.int32),
            pltpu.VMEM((AJ, 128), jnp.int32),
            pltpu.VMEM((J, 128), jnp.int32),
            pltpu.VMEM((NBUF, 128, D), jnp.float32),
            pltpu.VMEM((AJ, 128, D), jnp.float32),
        ] + [pltpu.SemaphoreType.DMA] * (2 * NBUF + 1),
    )
    out = call(tok1d, obs_table, act_table)
    return out.reshape(B, SPAD, 2 * D)[:, :S, :D]
